# baseline (speedup 1.0000x reference)
import functools

import jax
import jax.numpy as jnp
from jax import lax
from jax.experimental import pallas as pl
from jax.experimental.pallas import tpu as pltpu
from jax.experimental.pallas import tpu_sc as plsc

VOCAB = 1000000
MAXLEN = 200
EMBED_DIM = 64
BATCH = 4096

NC = 2
NS = 16
NW = NC * NS
TOKENS = BATCH * MAXLEN
PER_W = TOKENS // NW
CHUNK = 128
N_CHUNKS = PER_W // CHUNK
LANES = 16
NBUF = 4
OUT_ROWS = CHUNK * EMBED_DIM // 128


def _body(idx_hbm, table_hbm, pos_hbm, out_hbm, idx_v, rows_v, outb_v, pos2_v,
          *sems):
    gsems = sems[:NBUF]
    osems = sems[NBUF:]
    wid = lax.axis_index("s") * NC + lax.axis_index("c")
    row0 = wid * (PER_W // CHUNK)
    orow0 = wid * (PER_W * EMBED_DIM // 128)

    pltpu.sync_copy(idx_hbm.at[pl.ds(row0, N_CHUNKS)], idx_v)
    pltpu.sync_copy(pos_hbm, pos2_v.at[pl.ds(0, MAXLEN)])
    pltpu.sync_copy(pos_hbm, pos2_v.at[pl.ds(MAXLEN, MAXLEN)])

    def gather(i, s):
        return pltpu.make_async_copy(
            table_hbm.at[idx_v.at[i]],
            rows_v.at[s],
            gsems[s])

    def out_copy(i, s):
        off = pl.multiple_of(orow0 + i * OUT_ROWS, OUT_ROWS)
        return pltpu.make_async_copy(
            outb_v.at[s],
            out_hbm.at[pl.ds(off, OUT_ROWS)],
            osems[s])

    for s in range(NBUF - 1):
        gather(s, s).start()

    def chunk_body(i0, carry):
        for s in range(NBUF):
            i = i0 * NBUF + s
            sp = (s + NBUF - 1) % NBUF
            pf = i + NBUF - 1

            @pl.when(pf < N_CHUNKS)
            def _():
                gather(pf, sp).start()

            gather(i, s).wait()

            @pl.when(i >= NBUF)
            def _():
                out_copy(i - NBUF, s).wait()

            ph = lax.rem(i * CHUNK, MAXLEN)

            def row_body(jp, c2):
                j = jp * 2
                for r in range(2):
                    for c in range(EMBED_DIM // LANES):
                        src = pl.ds(c * LANES, LANES)
                        dst = pl.ds(r * EMBED_DIM + c * LANES, LANES)
                        outb_v[s, jp, dst] = (
                            rows_v[s, j + r, src] + pos2_v[ph + j + r, src])
                return c2

            lax.fori_loop(0, CHUNK // 2, row_body, 0)
            out_copy(i, s).start()
        return carry

    lax.fori_loop(0, N_CHUNKS // NBUF, chunk_body, 0)
    for s in range(NBUF):
        out_copy(N_CHUNKS - NBUF + s, s).wait()


def kernel(inputs, token_table, pos_table):
    idx = jnp.reshape(inputs.astype(jnp.int32), (TOKENS // 128, 128))
    mesh = plsc.VectorSubcoreMesh(core_axis_name="c", subcore_axis_name="s")
    fn = functools.partial(
        pl.kernel,
        mesh=mesh,
        compiler_params=pltpu.CompilerParams(use_tc_tiling_on_sc=False),
        out_type=jax.ShapeDtypeStruct((TOKENS * EMBED_DIM // 128, 128),
                                      jnp.float32),
        scratch_types=[
            pltpu.VMEM((N_CHUNKS, CHUNK), jnp.int32),
            pltpu.VMEM((NBUF, CHUNK, EMBED_DIM), jnp.float32),
            pltpu.VMEM((NBUF, OUT_ROWS, 128), jnp.float32),
            pltpu.VMEM((2 * MAXLEN, EMBED_DIM), jnp.float32),
        ] + [pltpu.SemaphoreType.DMA] * (2 * NBUF),
    )(_body)
    out = fn(idx, token_table, pos_table)
    return jnp.reshape(out, (BATCH, MAXLEN, EMBED_DIM))
